# baseline (speedup 1.0000x reference)
import jax
import jax.numpy as jnp
from jax.experimental import pallas as pl


def _mlp_kernel(x_ref, w1_ref, b1_ref, gamma_ref, beta_ref, mean_ref,
                var_ref, w2_ref, b2_ref, o_ref):
    h = jnp.dot(x_ref[...], w1_ref[...], preferred_element_type=jnp.float32)
    h = jnp.maximum(h + b1_ref[...], 0.0)
    scale = gamma_ref[...] * jax.lax.rsqrt(var_ref[...] + 1e-5)
    shift = beta_ref[...] - mean_ref[...] * scale
    h = (h * scale + shift).astype(jnp.bfloat16)
    o_ref[...] = (jnp.dot(h, w2_ref[...].astype(jnp.bfloat16),
                          preferred_element_type=jnp.float32)
                  + b2_ref[...])


def kernel(x, W1, b1, gamma, beta, running_mean, running_var, W2, b2):
    B, L, D_in = x.shape
    D_hid = W1.shape[1]
    D_out = W2.shape[1]
    M = B * L
    TM = 2048
    x2 = x.reshape(M, D_in).astype(jnp.bfloat16)
    W1 = W1.astype(jnp.bfloat16)
    vec = lambda v: v.reshape(1, -1)

    out = pl.pallas_call(
        _mlp_kernel,
        grid=(M // TM,),
        in_specs=[
            pl.BlockSpec((TM, D_in), lambda i: (i, 0)),
            pl.BlockSpec((D_in, D_hid), lambda i: (0, 0)),
            pl.BlockSpec((1, D_hid), lambda i: (0, 0)),
            pl.BlockSpec((1, D_hid), lambda i: (0, 0)),
            pl.BlockSpec((1, D_hid), lambda i: (0, 0)),
            pl.BlockSpec((1, D_hid), lambda i: (0, 0)),
            pl.BlockSpec((1, D_hid), lambda i: (0, 0)),
            pl.BlockSpec((D_hid, D_out), lambda i: (0, 0)),
            pl.BlockSpec((1, D_out), lambda i: (0, 0)),
        ],
        out_specs=pl.BlockSpec((TM, D_out), lambda i: (i, 0)),
        out_shape=jax.ShapeDtypeStruct((M, D_out), jnp.float32),
    )(x2, W1, vec(b1), vec(gamma), vec(beta), vec(running_mean),
      vec(running_var), W2, vec(b2))
    return out.reshape(B, L, D_out)


# speedup vs baseline: 1.3625x; 1.3625x over previous
import jax
import jax.numpy as jnp
from jax.experimental import pallas as pl


def _mlp_kernel(x_ref, w1_ref, b1_ref, gamma_ref, beta_ref, mean_ref,
                var_ref, w2_ref, b2_ref, o_ref):
    h = jnp.dot(x_ref[...].astype(jnp.bfloat16),
                w1_ref[...].astype(jnp.bfloat16),
                preferred_element_type=jnp.float32)
    h = jnp.maximum(h + b1_ref[...], 0.0)
    scale = gamma_ref[...] * jax.lax.rsqrt(var_ref[...] + 1e-5)
    shift = beta_ref[...] - mean_ref[...] * scale
    h = (h * scale + shift).astype(jnp.bfloat16)
    o_ref[...] = (jnp.dot(h, w2_ref[...].astype(jnp.bfloat16),
                          preferred_element_type=jnp.float32)
                  + b2_ref[...])


def kernel(x, W1, b1, gamma, beta, running_mean, running_var, W2, b2):
    B, L, D_in = x.shape
    D_hid = W1.shape[1]
    D_out = W2.shape[1]
    M = B * L
    TM = 2048
    x2 = x.reshape(M, D_in)
    vec = lambda v: v.reshape(1, -1)

    out = pl.pallas_call(
        _mlp_kernel,
        grid=(M // TM,),
        in_specs=[
            pl.BlockSpec((TM, D_in), lambda i: (i, 0)),
            pl.BlockSpec((D_in, D_hid), lambda i: (0, 0)),
            pl.BlockSpec((1, D_hid), lambda i: (0, 0)),
            pl.BlockSpec((1, D_hid), lambda i: (0, 0)),
            pl.BlockSpec((1, D_hid), lambda i: (0, 0)),
            pl.BlockSpec((1, D_hid), lambda i: (0, 0)),
            pl.BlockSpec((1, D_hid), lambda i: (0, 0)),
            pl.BlockSpec((D_hid, D_out), lambda i: (0, 0)),
            pl.BlockSpec((1, D_out), lambda i: (0, 0)),
        ],
        out_specs=pl.BlockSpec((TM, D_out), lambda i: (i, 0)),
        out_shape=jax.ShapeDtypeStruct((M, D_out), jnp.float32),
    )(x2, W1, vec(b1), vec(gamma), vec(beta), vec(running_mean),
      vec(running_var), W2, vec(b2))
    return out.reshape(B, L, D_out)


# speedup vs baseline: 1.3724x; 1.0073x over previous
import jax
import jax.numpy as jnp
from jax.experimental import pallas as pl
from jax.experimental.pallas import tpu as pltpu


def _mlp_kernel(x_ref, w1_ref, b1_ref, gamma_ref, beta_ref, mean_ref,
                var_ref, w2_ref, b2_ref, o_ref):
    h = jnp.dot(x_ref[...].astype(jnp.bfloat16),
                w1_ref[...].astype(jnp.bfloat16),
                preferred_element_type=jnp.float32)
    h = jnp.maximum(h + b1_ref[...], 0.0)
    scale = gamma_ref[...] * jax.lax.rsqrt(var_ref[...] + 1e-5)
    shift = beta_ref[...] - mean_ref[...] * scale
    h = (h * scale + shift).astype(jnp.bfloat16)
    o_ref[...] = (jnp.dot(h, w2_ref[...].astype(jnp.bfloat16),
                          preferred_element_type=jnp.float32)
                  + b2_ref[...])


def kernel(x, W1, b1, gamma, beta, running_mean, running_var, W2, b2):
    B, L, D_in = x.shape
    D_hid = W1.shape[1]
    D_out = W2.shape[1]
    M = B * L
    TM = 2048
    x2 = x.reshape(M, D_in)
    vec = lambda v: v.reshape(1, -1)

    out = pl.pallas_call(
        _mlp_kernel,
        grid=(M // TM,),
        in_specs=[
            pl.BlockSpec((TM, D_in), lambda i: (i, 0)),
            pl.BlockSpec((D_in, D_hid), lambda i: (0, 0)),
            pl.BlockSpec((1, D_hid), lambda i: (0, 0)),
            pl.BlockSpec((1, D_hid), lambda i: (0, 0)),
            pl.BlockSpec((1, D_hid), lambda i: (0, 0)),
            pl.BlockSpec((1, D_hid), lambda i: (0, 0)),
            pl.BlockSpec((1, D_hid), lambda i: (0, 0)),
            pl.BlockSpec((D_hid, D_out), lambda i: (0, 0)),
            pl.BlockSpec((1, D_out), lambda i: (0, 0)),
        ],
        out_specs=pl.BlockSpec((TM, D_out), lambda i: (i, 0)),
        out_shape=jax.ShapeDtypeStruct((M, D_out), jnp.float32),
        compiler_params=pltpu.CompilerParams(
            dimension_semantics=("parallel",)),
    )(x2, W1, vec(b1), vec(gamma), vec(beta), vec(running_mean),
      vec(running_var), W2, vec(b2))
    return out.reshape(B, L, D_out)
